# baseline (speedup 1.0000x reference)
import functools

import jax
import jax.numpy as jnp
from jax import lax
from jax.experimental import pallas as pl
from jax.experimental.pallas import tpu as pltpu
from jax.experimental.pallas import tpu_sc as plsc

DICT_SIZE = 1000000
EMBED_DIM = 64
BATCH = 16384
HIST = 20
TOTAL = BATCH * HIST

NUM_CORES = 2
NUM_SUBCORES = 16
NW = NUM_CORES * NUM_SUBCORES
PER_W = TOTAL // NW
CHUNK = 128
NCHUNK = PER_W // CHUNK
NSLOT = 8

_mesh = plsc.VectorSubcoreMesh(core_axis_name="c", subcore_axis_name="s")


@functools.partial(
    pl.kernel,
    out_type=jax.ShapeDtypeStruct((TOTAL, EMBED_DIM), jnp.float32),
    mesh=_mesh,
    scratch_types=[
        pltpu.VMEM((NCHUNK, CHUNK), jnp.int32),
        pltpu.VMEM((NSLOT, CHUNK, EMBED_DIM), jnp.float32),
        pltpu.SemaphoreType.DMA,
        [pltpu.SemaphoreType.DMA] * NSLOT,
        [pltpu.SemaphoreType.DMA] * NSLOT,
    ],
    compiler_params=pltpu.CompilerParams(use_tc_tiling_on_sc=False),
)
def _emb_lookup(idx_hbm, table_hbm, out_hbm, idx_v, rows_v, isem, gsems, wsems):
    wid = lax.axis_index("s") * NUM_CORES + lax.axis_index("c")
    base = wid * PER_W

    pltpu.async_copy(idx_hbm.at[pl.ds(wid * NCHUNK, NCHUNK)], idx_v, isem).wait()

    for b in range(NSLOT):
        pltpu.async_copy(table_hbm.at[idx_v.at[b]], rows_v.at[b], gsems[b])

    @pl.loop(0, NCHUNK, step=NSLOT)
    def _group(g):
        for b in range(NSLOT):
            j = g + b
            pltpu.make_async_copy(
                table_hbm.at[idx_v.at[b]], rows_v.at[b], gsems[b]
            ).wait()
            pltpu.async_copy(
                rows_v.at[b],
                out_hbm.at[pl.ds(base + j * CHUNK, CHUNK)],
                wsems[b],
            )
        for b in range(NSLOT):
            jn = g + NSLOT + b
            pltpu.make_async_copy(
                rows_v.at[b],
                out_hbm.at[pl.ds(base, CHUNK)],
                wsems[b],
            ).wait()

            @pl.when(jn < NCHUNK)
            def _():
                pltpu.async_copy(table_hbm.at[idx_v.at[jn]], rows_v.at[b], gsems[b])


def kernel(x, weight):
    idx2d = x.astype(jnp.int32).reshape(TOTAL // CHUNK, CHUNK)
    out = _emb_lookup(idx2d, weight)
    return out.reshape(BATCH, HIST, EMBED_DIM)


# speedup vs baseline: 1.0005x; 1.0005x over previous
import functools

import jax
import jax.numpy as jnp
from jax import lax
from jax.experimental import pallas as pl
from jax.experimental.pallas import tpu as pltpu
from jax.experimental.pallas import tpu_sc as plsc

DICT_SIZE = 1000000
EMBED_DIM = 64
BATCH = 16384
HIST = 20
TOTAL = BATCH * HIST

NUM_CORES = 2
NUM_SUBCORES = 16
NW = NUM_CORES * NUM_SUBCORES
PER_W = TOTAL // NW
CHUNK = 128
NCHUNK = PER_W // CHUNK
NSLOT = 10
LOOKAHEAD = 5

_mesh = plsc.VectorSubcoreMesh(core_axis_name="c", subcore_axis_name="s")


@functools.partial(
    pl.kernel,
    out_type=jax.ShapeDtypeStruct((TOTAL, EMBED_DIM), jnp.float32),
    mesh=_mesh,
    scratch_types=[
        pltpu.VMEM((NCHUNK, CHUNK), jnp.int32),
        pltpu.VMEM((NSLOT, CHUNK, EMBED_DIM), jnp.float32),
        pltpu.SemaphoreType.DMA,
        [pltpu.SemaphoreType.DMA] * NSLOT,
        [pltpu.SemaphoreType.DMA] * NSLOT,
    ],
    compiler_params=pltpu.CompilerParams(use_tc_tiling_on_sc=False),
)
def _emb_lookup(idx_hbm, table_hbm, out_hbm, idx_v, rows_v, isem, gsems, wsems):
    wid = lax.axis_index("s") * NUM_CORES + lax.axis_index("c")
    base = wid * PER_W

    pltpu.async_copy(idx_hbm.at[pl.ds(wid * NCHUNK, NCHUNK)], idx_v, isem).wait()

    for b in range(LOOKAHEAD):
        pltpu.async_copy(table_hbm.at[idx_v.at[b]], rows_v.at[b], gsems[b])

    @pl.loop(0, NCHUNK, step=NSLOT)
    def _group(g):
        for b in range(NSLOT):
            j = g + b
            jn = j + LOOKAHEAD
            bn = (b + LOOKAHEAD) % NSLOT

            @pl.when(jn < NCHUNK)
            def _():
                @pl.when(jn >= NSLOT)
                def _():
                    pltpu.make_async_copy(
                        rows_v.at[bn],
                        out_hbm.at[pl.ds(base, CHUNK)],
                        wsems[bn],
                    ).wait()

                pltpu.async_copy(table_hbm.at[idx_v.at[jn]], rows_v.at[bn], gsems[bn])

            pltpu.make_async_copy(
                table_hbm.at[idx_v.at[b]], rows_v.at[b], gsems[b]
            ).wait()
            pltpu.async_copy(
                rows_v.at[b],
                out_hbm.at[pl.ds(base + j * CHUNK, CHUNK)],
                wsems[b],
            )

    for b in range(NSLOT):
        pltpu.make_async_copy(
            rows_v.at[b], out_hbm.at[pl.ds(base, CHUNK)], wsems[b]
        ).wait()


def kernel(x, weight):
    idx2d = x.astype(jnp.int32).reshape(TOTAL // CHUNK, CHUNK)
    out = _emb_lookup(idx2d, weight)
    return out.reshape(BATCH, HIST, EMBED_DIM)


# speedup vs baseline: 1.0419x; 1.0413x over previous
import functools

import jax
import jax.numpy as jnp
from jax import lax
from jax.experimental import pallas as pl
from jax.experimental.pallas import tpu as pltpu
from jax.experimental.pallas import tpu_sc as plsc

DICT_SIZE = 1000000
EMBED_DIM = 64
BATCH = 16384
HIST = 20
TOTAL = BATCH * HIST

NUM_CORES = 2
NUM_SUBCORES = 16
NW = NUM_CORES * NUM_SUBCORES
PER_W = TOTAL // NW
CHUNK = 128
NCHUNK = PER_W // CHUNK
PADW = 128
NSLOT = 5
LOOKAHEAD = 2

_mesh = plsc.VectorSubcoreMesh(core_axis_name="c", subcore_axis_name="s")


@functools.partial(
    pl.kernel,
    out_type=jax.ShapeDtypeStruct((TOTAL, EMBED_DIM), jnp.float32),
    mesh=_mesh,
    scratch_types=[
        pltpu.VMEM((NCHUNK, CHUNK), jnp.int32),
        pltpu.VMEM((NSLOT, CHUNK, PADW), jnp.float32),
        pltpu.SemaphoreType.DMA,
        [pltpu.SemaphoreType.DMA] * NSLOT,
        [pltpu.SemaphoreType.DMA] * NSLOT,
    ],
    compiler_params=pltpu.CompilerParams(use_tc_tiling_on_sc=False),
)
def _emb_lookup(idx_hbm, table_hbm, out_hbm, idx_v, rows_v, isem, gsems, wsems):
    wid = lax.axis_index("s") * NUM_CORES + lax.axis_index("c")
    base = wid * PER_W

    pltpu.async_copy(idx_hbm.at[pl.ds(wid * NCHUNK, NCHUNK)], idx_v, isem).wait()

    for b in range(LOOKAHEAD):
        pltpu.async_copy(table_hbm.at[idx_v.at[b]], rows_v.at[b], gsems[b])

    @pl.loop(0, NCHUNK, step=NSLOT)
    def _group(g):
        for b in range(NSLOT):
            j = g + b
            jn = j + LOOKAHEAD
            bn = (b + LOOKAHEAD) % NSLOT

            @pl.when(jn < NCHUNK)
            def _():
                @pl.when(jn >= NSLOT)
                def _():
                    pltpu.make_async_copy(
                        rows_v.at[bn],
                        out_hbm.at[pl.ds(base, CHUNK)],
                        wsems[bn],
                    ).wait()

                pltpu.async_copy(table_hbm.at[idx_v.at[jn]], rows_v.at[bn], gsems[bn])

            pltpu.make_async_copy(
                table_hbm.at[idx_v.at[b]], rows_v.at[b], gsems[b]
            ).wait()
            pltpu.async_copy(
                rows_v.at[b, :, pl.ds(0, EMBED_DIM)],
                out_hbm.at[pl.ds(base + j * CHUNK, CHUNK)],
                wsems[b],
            )

    for b in range(NSLOT):
        pltpu.make_async_copy(
            rows_v.at[b, :, pl.ds(0, EMBED_DIM)],
            out_hbm.at[pl.ds(base, CHUNK)],
            wsems[b],
        ).wait()


def kernel(x, weight):
    idx2d = x.astype(jnp.int32).reshape(TOTAL // CHUNK, CHUNK)
    wpad = jnp.pad(weight, ((0, 0), (0, PADW - EMBED_DIM)))
    out = _emb_lookup(idx2d, wpad)
    return out.reshape(BATCH, HIST, EMBED_DIM)
